# baseline (speedup 1.0000x reference)
import jax
import jax.numpy as jnp
from jax.experimental import pallas as pl
from jax.experimental.pallas import tpu as pltpu

_C = 1.0
_MIN_NORM = 1e-15
_BALL_EPS = 4e-3

_DOT_DIMS = (((1,), (0,)), ((), ()))

_IB = 40
_NSLOT = 4


def _row_norm(v, ones):
    sq = jax.lax.dot_general(v * v, ones, dimension_numbers=_DOT_DIMS,
                             preferred_element_type=jnp.float32)
    return jnp.maximum(jnp.sqrt(sq), _MIN_NORM)


def _slab_copy(adj_hbm, buf_ref, sems, block, slot):
    return pltpu.make_async_copy(
        adj_hbm.at[pl.ds(block * _IB, _IB), :],
        buf_ref.at[slot],
        sems.at[slot],
    )


def _hyp_agg_body(x_ref, adj_hbm, out_ref, xt_ref, buf_ref, sems):
    i = pl.program_id(0)
    ni = pl.num_programs(0)
    d = x_ref.shape[1]
    ones = jnp.ones((d, d), jnp.float32)

    @pl.when(i == 0)
    def _start():
        for s in range(_NSLOT):
            _slab_copy(adj_hbm, buf_ref, sems, s, s).start()
        xv = x_ref[...]
        sq = jax.lax.dot_general(xv * xv, ones, dimension_numbers=_DOT_DIMS,
                                 preferred_element_type=jnp.float32)
        rnorm = jax.lax.rsqrt(jnp.maximum(sq, _MIN_NORM * _MIN_NORM))
        norm = sq * rnorm
        t = jnp.clip(norm, -1.0 + 1e-7, 1.0 - 1e-7)
        artanh = 0.5 * jnp.log((1.0 + t) / (1.0 - t))
        xt_ref[...] = xv * (artanh * rnorm)

    slot = jax.lax.rem(i, _NSLOT)
    _slab_copy(adj_hbm, buf_ref, sems, i, slot).wait()
    u = jax.lax.dot_general(
        buf_ref[slot], xt_ref[...], dimension_numbers=_DOT_DIMS,
        preferred_element_type=jnp.float32,
    )
    norm = _row_norm(u, ones)
    t = jnp.tanh(norm)
    scale = jnp.minimum(t, 1.0 - _BALL_EPS) / norm
    out_ref[...] = u * scale

    @pl.when(i + _NSLOT < ni)
    def _refill():
        _slab_copy(adj_hbm, buf_ref, sems, i + _NSLOT, slot).start()


def kernel(x, adj):
    n, d = x.shape
    ni = n // _IB
    return pl.pallas_call(
        _hyp_agg_body,
        grid=(ni,),
        in_specs=[
            pl.BlockSpec((n, d), lambda i: (0, 0)),
            pl.BlockSpec(memory_space=pl.ANY),
        ],
        out_specs=pl.BlockSpec((_IB, d), lambda i: (i, 0)),
        out_shape=jax.ShapeDtypeStruct((n, d), jnp.float32),
        scratch_shapes=[
            pltpu.VMEM((n, d), jnp.float32),
            pltpu.VMEM((_NSLOT, _IB, n), jnp.float32),
            pltpu.SemaphoreType.DMA((_NSLOT,)),
        ],
        compiler_params=pltpu.CompilerParams(
            dimension_semantics=("arbitrary",),
        ),
    )(x, adj)


# speedup vs baseline: 1.5305x; 1.5305x over previous
import jax
import jax.numpy as jnp
from jax.experimental import pallas as pl
from jax.experimental.pallas import tpu as pltpu

_C = 1.0
_MIN_NORM = 1e-15
_BALL_EPS = 4e-3

_DOT_DIMS = (((1,), (0,)), ((), ()))

_IB = 80
_NSLOT = 5


def _row_norm(v, ones):
    sq = jax.lax.dot_general(v * v, ones, dimension_numbers=_DOT_DIMS,
                             preferred_element_type=jnp.float32)
    return jnp.maximum(jnp.sqrt(sq), _MIN_NORM)


def _slab_copy(adj_hbm, buf_ref, sems, block, slot):
    return pltpu.make_async_copy(
        adj_hbm.at[pl.ds(block * _IB, _IB), :],
        buf_ref.at[slot],
        sems.at[slot],
    )


def _hyp_agg_body(x_ref, adj_hbm, out_ref, xt_ref, buf_ref, sems):
    i = pl.program_id(0)
    ni = pl.num_programs(0)
    d = x_ref.shape[1]
    ones = jnp.ones((d, d), jnp.float32)

    @pl.when(i == 0)
    def _start():
        for s in range(_NSLOT):
            _slab_copy(adj_hbm, buf_ref, sems, s, s).start()
        xv = x_ref[...]
        sq = jax.lax.dot_general(xv * xv, ones, dimension_numbers=_DOT_DIMS,
                                 preferred_element_type=jnp.float32)
        rnorm = jax.lax.rsqrt(jnp.maximum(sq, _MIN_NORM * _MIN_NORM))
        norm = sq * rnorm
        t = jnp.clip(norm, -1.0 + 1e-7, 1.0 - 1e-7)
        artanh = 0.5 * jnp.log((1.0 + t) / (1.0 - t))
        xt_ref[...] = xv * (artanh * rnorm)

    slot = jax.lax.rem(i, _NSLOT)
    _slab_copy(adj_hbm, buf_ref, sems, i, slot).wait()
    u = jax.lax.dot_general(
        buf_ref[slot], xt_ref[...], dimension_numbers=_DOT_DIMS,
        preferred_element_type=jnp.float32,
    )
    norm = _row_norm(u, ones)
    t = jnp.tanh(norm)
    scale = jnp.minimum(t, 1.0 - _BALL_EPS) / norm
    out_ref[...] = u * scale

    @pl.when(i + _NSLOT < ni)
    def _refill():
        _slab_copy(adj_hbm, buf_ref, sems, i + _NSLOT, slot).start()


def kernel(x, adj):
    n, d = x.shape
    ni = n // _IB
    return pl.pallas_call(
        _hyp_agg_body,
        grid=(ni,),
        in_specs=[
            pl.BlockSpec((n, d), lambda i: (0, 0)),
            pl.BlockSpec(memory_space=pl.ANY),
        ],
        out_specs=pl.BlockSpec((_IB, d), lambda i: (i, 0)),
        out_shape=jax.ShapeDtypeStruct((n, d), jnp.float32),
        scratch_shapes=[
            pltpu.VMEM((n, d), jnp.float32),
            pltpu.VMEM((_NSLOT, _IB, n), jnp.float32),
            pltpu.SemaphoreType.DMA((_NSLOT,)),
        ],
        compiler_params=pltpu.CompilerParams(
            dimension_semantics=("arbitrary",),
        ),
    )(x, adj)


# speedup vs baseline: 1.5478x; 1.0113x over previous
import jax
import jax.numpy as jnp
from jax.experimental import pallas as pl
from jax.experimental.pallas import tpu as pltpu

_C = 1.0
_MIN_NORM = 1e-15
_BALL_EPS = 4e-3

_DOT_DIMS = (((1,), (0,)), ((), ()))

_IB = 80
_NSLOT = 4


def _row_norm(v, ones):
    sq = jax.lax.dot_general(v * v, ones, dimension_numbers=_DOT_DIMS,
                             preferred_element_type=jnp.float32)
    return jnp.maximum(jnp.sqrt(sq), _MIN_NORM)


def _slab_copy(adj_hbm, buf_ref, sems, block, slot):
    return pltpu.make_async_copy(
        adj_hbm.at[pl.ds(block * _IB, _IB), :],
        buf_ref.at[slot],
        sems.at[slot],
    )


def _hyp_agg_body(x_ref, adj_hbm, out_ref, xt_ref, buf_ref, sems):
    i = pl.program_id(0)
    ni = pl.num_programs(0)
    d = x_ref.shape[1]
    ones = jnp.ones((d, d), jnp.float32)

    @pl.when(i == 0)
    def _start():
        for s in range(_NSLOT):
            _slab_copy(adj_hbm, buf_ref, sems, s, s).start()
        xv = x_ref[...]
        sq = jax.lax.dot_general(xv * xv, ones, dimension_numbers=_DOT_DIMS,
                                 preferred_element_type=jnp.float32)
        rnorm = jax.lax.rsqrt(jnp.maximum(sq, _MIN_NORM * _MIN_NORM))
        norm = sq * rnorm
        t = jnp.clip(norm, -1.0 + 1e-7, 1.0 - 1e-7)
        artanh = 0.5 * jnp.log((1.0 + t) / (1.0 - t))
        xt_ref[...] = xv * (artanh * rnorm)

    slot = jax.lax.rem(i, _NSLOT)
    _slab_copy(adj_hbm, buf_ref, sems, i, slot).wait()
    u = jax.lax.dot_general(
        buf_ref[slot], xt_ref[...], dimension_numbers=_DOT_DIMS,
        preferred_element_type=jnp.float32,
    )
    norm = _row_norm(u, ones)
    t = jnp.tanh(norm)
    scale = jnp.minimum(t, 1.0 - _BALL_EPS) / norm
    out_ref[...] = u * scale

    @pl.when(i + _NSLOT < ni)
    def _refill():
        _slab_copy(adj_hbm, buf_ref, sems, i + _NSLOT, slot).start()


def kernel(x, adj):
    n, d = x.shape
    ni = n // _IB
    return pl.pallas_call(
        _hyp_agg_body,
        grid=(ni,),
        in_specs=[
            pl.BlockSpec((n, d), lambda i: (0, 0)),
            pl.BlockSpec(memory_space=pl.ANY),
        ],
        out_specs=pl.BlockSpec((_IB, d), lambda i: (i, 0)),
        out_shape=jax.ShapeDtypeStruct((n, d), jnp.float32),
        scratch_shapes=[
            pltpu.VMEM((n, d), jnp.float32),
            pltpu.VMEM((_NSLOT, _IB, n), jnp.float32),
            pltpu.SemaphoreType.DMA((_NSLOT,)),
        ],
        compiler_params=pltpu.CompilerParams(
            dimension_semantics=("arbitrary",),
        ),
    )(x, adj)
